# Initial kernel scaffold; baseline (speedup 1.0000x reference)
import functools

import jax
import jax.numpy as jnp
from jax import lax
from jax.experimental import pallas as pl
from jax.experimental.pallas import tpu as pltpu
from jax.experimental.pallas import tpu_sc as plsc

B = 8
N = 8192
G = 512
K = 32
D = 128
GB = 128
XP = 16


def _knn_body(xyzT_ref, c_ref, out_ref):
    b = pl.program_id(0)
    x = xyzT_ref[0]
    x0 = x[0:1, :]
    x1 = x[1:2, :]
    x2 = x[2:3, :]
    c = c_ref[0]
    c0 = c[:, 0:1]
    c1 = c[:, 1:2]
    c2 = c[:, 2:3]
    m = (c0 * x0 + c1 * x1) + c2 * x2
    sqc = (c0 * c0 + c1 * c1) + c2 * c2
    sqx = (x0 * x0 + x1 * x1) + x2 * x2
    d = ((-2.0 * m) + sqc) + sqx
    iota = lax.broadcasted_iota(jnp.int32, (GB, N), 1)
    cols = []
    for _ in range(K):
        mn = jnp.min(d, axis=1, keepdims=True)
        am = jnp.min(jnp.where(d == mn, iota, N), axis=1, keepdims=True)
        cols.append(am)
        d = jnp.where(iota == am, jnp.float32(jnp.inf), d)
    out_ref[0] = jnp.concatenate(cols, axis=1) + b * N


def _knn_topk(xyzT, center):
    return pl.pallas_call(
        _knn_body,
        grid=(B, G // GB),
        in_specs=[
            pl.BlockSpec((1, 3, N), lambda b, g: (b, 0, 0)),
            pl.BlockSpec((1, GB, 3), lambda b, g: (b, g, 0)),
        ],
        out_specs=pl.BlockSpec((1, GB, K), lambda b, g: (b, g, 0)),
        out_shape=jax.ShapeDtypeStruct((B, G, K), jnp.int32),
    )(xyzT, center)


R = B * G * K
NW = 32
RPW = R // NW
C = 128


def _sc_gather(idx_flat, emb_flat, xyz_pad):
    mesh = plsc.VectorSubcoreMesh(core_axis_name="c", subcore_axis_name="s")

    @functools.partial(
        pl.kernel,
        mesh=mesh,
        out_type=(
            jax.ShapeDtypeStruct((R, D), jnp.float32),
            jax.ShapeDtypeStruct((R, XP), jnp.float32),
        ),
        scratch_types=[
            pltpu.VMEM((C,), jnp.int32),
            pltpu.VMEM((C, D), jnp.float32),
            pltpu.VMEM((C, XP), jnp.float32),
            pltpu.SemaphoreType.DMA,
            pltpu.SemaphoreType.DMA,
        ],
    )
    def gk(idx_hbm, emb_hbm, xyzp_hbm, eout, xout, idx_v, er_v, xr_v, s1, s2):
        wid = lax.axis_index("s") * 2 + lax.axis_index("c")

        def body(t, carry):
            base = wid * RPW + t * C
            pltpu.sync_copy(idx_hbm.at[pl.ds(base, C)], idx_v)
            ce = pltpu.async_copy(emb_hbm.at[idx_v], er_v, s1)
            cx = pltpu.async_copy(xyzp_hbm.at[idx_v], xr_v, s2)
            ce.wait()
            cx.wait()
            pltpu.sync_copy(er_v, eout.at[pl.ds(base, C)])
            pltpu.sync_copy(xr_v, xout.at[pl.ds(base, C)])
            return carry

        lax.fori_loop(0, RPW // C, body, 0)

    return gk(idx_flat, emb_flat, xyz_pad)


def kernel(xyz, emb):
    perm = jax.random.permutation(jax.random.key(42), N)[:G]
    center = jnp.take(xyz, perm, axis=1)
    xyzT = jnp.transpose(xyz, (0, 2, 1))
    flat_idx = _knn_topk(xyzT, center)
    emb_flat = emb.reshape(B * N, D)
    xyz_pad = jnp.pad(xyz.reshape(B * N, 3), ((0, 0), (0, XP - 3)))
    emb_g_flat, xyzg_flat = _sc_gather(flat_idx.reshape(-1), emb_flat, xyz_pad)
    emb_g = emb_g_flat.reshape(B, G, K, D)
    neighborhood = xyzg_flat[:, :3].reshape(B, G, K, 3) - center[:, :, None, :]
    return (neighborhood, center, emb_g)


# speedup vs baseline: 3.4220x; 3.4220x over previous
import functools

import jax
import jax.numpy as jnp
from jax import lax
from jax.experimental import pallas as pl
from jax.experimental.pallas import tpu as pltpu
from jax.experimental.pallas import tpu_sc as plsc

B = 8
N = 8192
G = 512
K = 32
D = 128
GB = 128
XP = 16


def _knn_body(xyzT_ref, c_ref, out_ref, nbx_ref, nby_ref, nbz_ref):
    b = pl.program_id(0)
    x = xyzT_ref[0]
    x0 = x[0:1, :]
    x1 = x[1:2, :]
    x2 = x[2:3, :]
    c = c_ref[0]
    c0 = c[:, 0:1]
    c1 = c[:, 1:2]
    c2 = c[:, 2:3]
    c0b = c0.astype(jnp.bfloat16).astype(jnp.float32)
    c1b = c1.astype(jnp.bfloat16).astype(jnp.float32)
    c2b = c2.astype(jnp.bfloat16).astype(jnp.float32)
    x0b = x0.astype(jnp.bfloat16).astype(jnp.float32)
    x1b = x1.astype(jnp.bfloat16).astype(jnp.float32)
    x2b = x2.astype(jnp.bfloat16).astype(jnp.float32)
    m = (c0b * x0b + c1b * x1b) + c2b * x2b
    sqc = (c0 * c0 + c1 * c1) + c2 * c2
    sqx = (x0 * x0 + x1 * x1) + x2 * x2
    d = ((-2.0 * m) + sqc) + sqx
    iota = lax.broadcasted_iota(jnp.int32, (GB, N), 1)
    cols, nbx, nby, nbz = [], [], [], []
    for _ in range(K):
        mn = jnp.min(d, axis=1, keepdims=True)
        am = jnp.min(jnp.where(d == mn, iota, N), axis=1, keepdims=True)
        cols.append(am)
        sel = iota == am
        nbx.append(jnp.sum(jnp.where(sel, x0, 0.0), axis=1, keepdims=True) - c0)
        nby.append(jnp.sum(jnp.where(sel, x1, 0.0), axis=1, keepdims=True) - c1)
        nbz.append(jnp.sum(jnp.where(sel, x2, 0.0), axis=1, keepdims=True) - c2)
        d = jnp.where(sel, jnp.float32(jnp.inf), d)
    out_ref[0] = jnp.concatenate(cols, axis=1) + b * N
    nbx_ref[0] = jnp.concatenate(nbx, axis=1)
    nby_ref[0] = jnp.concatenate(nby, axis=1)
    nbz_ref[0] = jnp.concatenate(nbz, axis=1)


def _knn_topk(xyzT, center):
    return pl.pallas_call(
        _knn_body,
        grid=(B, G // GB),
        in_specs=[
            pl.BlockSpec((1, 3, N), lambda b, g: (b, 0, 0)),
            pl.BlockSpec((1, GB, 3), lambda b, g: (b, g, 0)),
        ],
        out_specs=[pl.BlockSpec((1, GB, K), lambda b, g: (b, g, 0))] * 4,
        out_shape=(
            jax.ShapeDtypeStruct((B, G, K), jnp.int32),
            jax.ShapeDtypeStruct((B, G, K), jnp.float32),
            jax.ShapeDtypeStruct((B, G, K), jnp.float32),
            jax.ShapeDtypeStruct((B, G, K), jnp.float32),
        ),
    )(xyzT, center)


R = B * G * K
NW = 32
RPW = R // NW
C = 128


def _sc_gather(idx_flat, emb_flat):
    mesh = plsc.VectorSubcoreMesh(core_axis_name="c", subcore_axis_name="s")

    @functools.partial(
        pl.kernel,
        mesh=mesh,
        out_type=jax.ShapeDtypeStruct((R, D), jnp.float32),
        scratch_types=[
            pltpu.VMEM((C,), jnp.int32),
            pltpu.VMEM((C, D), jnp.float32),
            pltpu.SemaphoreType.DMA,
        ],
    )
    def gk(idx_hbm, emb_hbm, eout, idx_v, er_v, s1):
        wid = lax.axis_index("s") * 2 + lax.axis_index("c")

        def body(t, carry):
            base = wid * RPW + t * C
            pltpu.sync_copy(idx_hbm.at[pl.ds(base, C)], idx_v)
            pltpu.async_copy(emb_hbm.at[idx_v], er_v, s1).wait()
            pltpu.sync_copy(er_v, eout.at[pl.ds(base, C)])
            return carry

        lax.fori_loop(0, RPW // C, body, 0)

    return gk(idx_flat, emb_flat)


def kernel(xyz, emb):
    perm = jax.random.permutation(jax.random.key(42), N)[:G]
    center = jnp.take(xyz, perm, axis=1)
    xyzT = jnp.transpose(xyz, (0, 2, 1))
    flat_idx, nbx, nby, nbz = _knn_topk(xyzT, center)
    emb_g_flat = _sc_gather(flat_idx.reshape(-1), emb.reshape(B * N, D))
    emb_g = emb_g_flat.reshape(B, G, K, D)
    neighborhood = jnp.stack([nbx, nby, nbz], axis=-1)
    return (neighborhood, center, emb_g)


# speedup vs baseline: 7.2756x; 2.1261x over previous
import functools

import jax
import jax.numpy as jnp
from jax import lax
from jax.experimental import pallas as pl
from jax.experimental.pallas import tpu as pltpu
from jax.experimental.pallas import tpu_sc as plsc

B = 8
N = 8192
G = 512
K = 32
D = 128
GB = 128


def _knn_body(xyzT_ref, c_ref, out_ref):
    b = pl.program_id(0)
    x = xyzT_ref[0]
    x0 = x[0:1, :]
    x1 = x[1:2, :]
    x2 = x[2:3, :]
    c = c_ref[0]
    c0 = c[:, 0:1]
    c1 = c[:, 1:2]
    c2 = c[:, 2:3]
    c0b = c0.astype(jnp.bfloat16).astype(jnp.float32)
    c1b = c1.astype(jnp.bfloat16).astype(jnp.float32)
    c2b = c2.astype(jnp.bfloat16).astype(jnp.float32)
    x0b = x0.astype(jnp.bfloat16).astype(jnp.float32)
    x1b = x1.astype(jnp.bfloat16).astype(jnp.float32)
    x2b = x2.astype(jnp.bfloat16).astype(jnp.float32)
    m = (c0b * x0b + c1b * x1b) + c2b * x2b
    sqc = (c0 * c0 + c1 * c1) + c2 * c2
    sqx = (x0 * x0 + x1 * x1) + x2 * x2
    d = ((-2.0 * m) + sqc) + sqx
    iota = lax.broadcasted_iota(jnp.int32, (GB, N), 1)
    cols = []
    for _ in range(K):
        am = jnp.argmin(d, axis=1).astype(jnp.int32)[:, None]
        cols.append(am)
        d = jnp.where(iota == am, jnp.float32(jnp.inf), d)
    out_ref[0] = jnp.concatenate(cols, axis=1) + b * N


def _knn_topk(xyzT, center):
    return pl.pallas_call(
        _knn_body,
        grid=(B, G // GB),
        in_specs=[
            pl.BlockSpec((1, 3, N), lambda b, g: (b, 0, 0)),
            pl.BlockSpec((1, GB, 3), lambda b, g: (b, g, 0)),
        ],
        out_specs=pl.BlockSpec((1, GB, K), lambda b, g: (b, g, 0)),
        out_shape=jax.ShapeDtypeStruct((B, G, K), jnp.int32),
    )(xyzT, center)


R = B * G * K
NW = 32
RPW = R // NW
R3 = 3 * R
EPW = R3 // NW
C = 128


def _sc_gather(idx_flat, idx3, emb_flat, xyz_cols):
    mesh = plsc.VectorSubcoreMesh(core_axis_name="c", subcore_axis_name="s")

    @functools.partial(
        pl.kernel,
        mesh=mesh,
        out_type=(
            jax.ShapeDtypeStruct((R, D), jnp.float32),
            jax.ShapeDtypeStruct((R3,), jnp.float32),
        ),
        scratch_types=[
            pltpu.VMEM((C,), jnp.int32),
            pltpu.VMEM((C, D), jnp.float32),
            pltpu.VMEM((C,), jnp.float32),
            pltpu.SemaphoreType.DMA,
        ],
    )
    def gk(idx_hbm, idx3_hbm, emb_hbm, xyz_hbm, eout, xout, idx_v, er_v, xr_v, s1):
        wid = lax.axis_index("s") * 2 + lax.axis_index("c")

        def ebody(t, carry):
            base = wid * RPW + t * C
            pltpu.sync_copy(idx_hbm.at[pl.ds(base, C)], idx_v)
            pltpu.async_copy(emb_hbm.at[idx_v], er_v, s1).wait()
            pltpu.sync_copy(er_v, eout.at[pl.ds(base, C)])
            return carry

        lax.fori_loop(0, RPW // C, ebody, 0)

        def xbody(t, carry):
            base = wid * EPW + t * C
            pltpu.sync_copy(idx3_hbm.at[pl.ds(base, C)], idx_v)
            pltpu.async_copy(xyz_hbm.at[idx_v], xr_v, s1).wait()
            pltpu.sync_copy(xr_v, xout.at[pl.ds(base, C)])
            return carry

        lax.fori_loop(0, EPW // C, xbody, 0)

    return gk(idx_flat, idx3, emb_flat, xyz_cols)


def kernel(xyz, emb):
    perm = jax.random.permutation(jax.random.key(42), N)[:G]
    center = jnp.take(xyz, perm, axis=1)
    xyzT = jnp.transpose(xyz, (0, 2, 1))
    flat_idx = _knn_topk(xyzT, center).reshape(-1)
    idx3 = jnp.concatenate([flat_idx, flat_idx + B * N, flat_idx + 2 * B * N])
    xyz_cols = jnp.transpose(xyz.reshape(B * N, 3)).reshape(-1)
    emb_g_flat, xg = _sc_gather(flat_idx, idx3, emb.reshape(B * N, D), xyz_cols)
    emb_g = emb_g_flat.reshape(B, G, K, D)
    nb = jnp.moveaxis(xg.reshape(3, B, G, K), 0, -1)
    neighborhood = nb - center[:, :, None, :]
    return (neighborhood, center, emb_g)


# speedup vs baseline: 7.5134x; 1.0327x over previous
import functools

import jax
import jax.numpy as jnp
from jax import lax
from jax.experimental import pallas as pl
from jax.experimental.pallas import tpu as pltpu
from jax.experimental.pallas import tpu_sc as plsc

B = 8
N = 8192
G = 512
K = 32
D = 128
GB = 128


def _knn_body(xyzT_ref, c_ref, out_ref):
    b = pl.program_id(0)
    x = xyzT_ref[0]
    x0 = x[0:1, :]
    x1 = x[1:2, :]
    x2 = x[2:3, :]
    c = c_ref[0]
    c0 = c[:, 0:1]
    c1 = c[:, 1:2]
    c2 = c[:, 2:3]
    c0b = c0.astype(jnp.bfloat16).astype(jnp.float32)
    c1b = c1.astype(jnp.bfloat16).astype(jnp.float32)
    c2b = c2.astype(jnp.bfloat16).astype(jnp.float32)
    x0b = x0.astype(jnp.bfloat16).astype(jnp.float32)
    x1b = x1.astype(jnp.bfloat16).astype(jnp.float32)
    x2b = x2.astype(jnp.bfloat16).astype(jnp.float32)
    m = (c0b * x0b + c1b * x1b) + c2b * x2b
    sqc = (c0 * c0 + c1 * c1) + c2 * c2
    sqx = (x0 * x0 + x1 * x1) + x2 * x2
    d = ((-2.0 * m) + sqc) + sqx
    iota = lax.broadcasted_iota(jnp.int32, (GB, N), 1)
    cols = []
    for _ in range(K):
        am = jnp.argmin(d, axis=1).astype(jnp.int32)[:, None]
        cols.append(am)
        d = jnp.where(iota == am, jnp.float32(jnp.inf), d)
    out_ref[0] = jnp.concatenate(cols, axis=1) + b * N


def _knn_topk(xyzT, center, nb):
    return pl.pallas_call(
        _knn_body,
        grid=(nb, G // GB),
        in_specs=[
            pl.BlockSpec((1, 3, N), lambda b, g: (b, 0, 0)),
            pl.BlockSpec((1, GB, 3), lambda b, g: (b, g, 0)),
        ],
        out_specs=pl.BlockSpec((1, GB, K), lambda b, g: (b, g, 0)),
        out_shape=jax.ShapeDtypeStruct((nb, G, K), jnp.int32),
    )(xyzT, center)


NW = 32
C = 128


def _sc_gather(idx_flat, idx3, emb_flat, xyz_cols):
    nr = idx_flat.shape[0]
    n3 = idx3.shape[0]
    rpw = nr // NW
    epw = n3 // NW
    mesh = plsc.VectorSubcoreMesh(core_axis_name="c", subcore_axis_name="s")

    @functools.partial(
        pl.kernel,
        mesh=mesh,
        out_type=(
            jax.ShapeDtypeStruct((nr, D), jnp.float32),
            jax.ShapeDtypeStruct((n3,), jnp.float32),
        ),
        scratch_types=[
            pltpu.VMEM((C,), jnp.int32),
            pltpu.VMEM((C, D), jnp.float32),
            pltpu.VMEM((C,), jnp.float32),
            pltpu.SemaphoreType.DMA,
        ],
    )
    def gk(idx_hbm, idx3_hbm, emb_hbm, xyz_hbm, eout, xout, idx_v, er_v, xr_v, s1):
        wid = lax.axis_index("s") * 2 + lax.axis_index("c")

        def ebody(t, carry):
            base = wid * rpw + t * C
            pltpu.sync_copy(idx_hbm.at[pl.ds(base, C)], idx_v)
            pltpu.async_copy(emb_hbm.at[idx_v], er_v, s1).wait()
            pltpu.sync_copy(er_v, eout.at[pl.ds(base, C)])
            return carry

        lax.fori_loop(0, rpw // C, ebody, 0)

        def xbody(t, carry):
            base = wid * epw + t * C
            pltpu.sync_copy(idx3_hbm.at[pl.ds(base, C)], idx_v)
            pltpu.async_copy(xyz_hbm.at[idx_v], xr_v, s1).wait()
            pltpu.sync_copy(xr_v, xout.at[pl.ds(base, C)])
            return carry

        lax.fori_loop(0, epw // C, xbody, 0)

    return gk(idx_flat, idx3, emb_flat, xyz_cols)


BC = 2
NB = B // BC


def kernel(xyz, emb):
    perm = jax.random.permutation(jax.random.key(42), N)[:G]
    center = jnp.take(xyz, perm, axis=1)
    xyzT = jnp.transpose(xyz, (0, 2, 1))
    emb_flat = emb.reshape(B * N, D)
    xyz_cols = jnp.transpose(xyz.reshape(B * N, 3)).reshape(-1)
    embs, nbs = [], []
    for cki in range(BC):
        b0 = cki * NB
        fi = _knn_topk(xyzT[b0:b0 + NB], center[b0:b0 + NB], NB).reshape(-1)
        fi = fi + b0 * N
        idx3 = jnp.concatenate([fi, fi + B * N, fi + 2 * B * N])
        eg, xg = _sc_gather(fi, idx3, emb_flat, xyz_cols)
        embs.append(eg.reshape(NB, G, K, D))
        nbs.append(jnp.moveaxis(xg.reshape(3, NB, G, K), 0, -1))
    emb_g = jnp.concatenate(embs, axis=0)
    neighborhood = jnp.concatenate(nbs, axis=0) - center[:, :, None, :]
    return (neighborhood, center, emb_g)
